# baseline (speedup 1.0000x reference)
import functools

import jax
import jax.numpy as jnp
import numpy as np
from jax import lax
from jax.experimental import pallas as pl
from jax.experimental.pallas import tpu as pltpu
from jax.experimental.pallas import tpu_sc as plsc

_P_TRAIN = 0.1
_ROWS, _COLS = 128, 8192

_MKEY = (928981903, 3453687069)


def _threefry2x32(k0, k1, x0, x1):
    u32 = np.uint32
    rot1 = (13, 15, 26, 6)
    rot2 = (17, 29, 16, 24)
    ks = (u32(k0), u32(k1), u32(k0) ^ u32(k1) ^ u32(0x1BD11BDA))
    x0 = x0 + ks[0]
    x1 = x1 + ks[1]

    def rotl(v, d):
        return (v << u32(d)) | (v >> u32(32 - d))

    for i in range(5):
        for r in rot1 if i % 2 == 0 else rot2:
            x0 = x0 + x1
            x1 = x0 ^ rotl(x1, r)
        x0 = x0 + ks[(i + 1) % 3]
        x1 = x1 + ks[(i + 2) % 3] + u32(i + 1)
    return x0, x1


def _draw_mask() -> np.ndarray:
    n = _ROWS * _COLS
    idx = np.arange(n, dtype=np.uint64)
    hi = (idx >> np.uint64(32)).astype(np.uint32)
    lo = (idx & np.uint64(0xFFFFFFFF)).astype(np.uint32)
    with np.errstate(over="ignore"):
        b1, b2 = _threefry2x32(_MKEY[0], _MKEY[1], hi, lo)
    bits64 = (b1.astype(np.uint64) << np.uint64(32)) | b2.astype(np.uint64)
    float_bits = (bits64 >> np.uint64(12)) | np.float64(1.0).view(np.uint64)
    f = float_bits.view(np.float64) - 1.0
    return (f < _P_TRAIN).reshape(_ROWS, _COLS)


def _build_gather_idx() -> np.ndarray:
    mask = _draw_mask()
    mask[:, -1] = False
    swap = np.roll(mask, 1, axis=1)
    mask = mask & ~swap
    swap = np.roll(mask, 1, axis=1)
    d = np.zeros((_ROWS, _COLS), np.int32)
    d[mask] = 1
    d[swap] = -1
    return (d + np.arange(_COLS, dtype=np.int32)[None, :]).astype(np.int32)


_IDX = _build_gather_idx()

_NC, _NS, _L = 2, 16, 16
_NW = _NC * _NS
_N = _ROWS * _COLS
_CHUNK = _N // _NW
_UNROLL = 8


def _sc_body(x_hbm, idx_hbm, out_hbm, x_v, i_v, o_v):
    wid = lax.axis_index("s") * _NC + lax.axis_index("c")
    base = wid * _CHUNK
    pltpu.sync_copy(x_hbm.at[pl.ds(base, _CHUNK)], x_v)
    pltpu.sync_copy(idx_hbm.at[pl.ds(base, _CHUNK)], i_v)

    def step(i, _):
        for u in range(_UNROLL):
            off = i * jnp.int32(_UNROLL * _L) + jnp.int32(u * _L)
            iv = i_v[pl.ds(off, _L)]
            o_v[pl.ds(off, _L)] = plsc.load_gather(x_v, [iv])
        return jnp.int32(0)

    lax.fori_loop(jnp.int32(0), jnp.int32(_CHUNK // (_L * _UNROLL)), step,
                  jnp.int32(0))
    pltpu.sync_copy(o_v, out_hbm.at[pl.ds(base, _CHUNK)])


_sc_swap = functools.partial(
    pl.kernel,
    mesh=plsc.VectorSubcoreMesh(core_axis_name="c", subcore_axis_name="s"),
    compiler_params=pltpu.CompilerParams(needs_layout_passes=False),
    out_type=jax.ShapeDtypeStruct((_N,), jnp.int32),
    scratch_types=[
        pltpu.VMEM((_CHUNK,), jnp.int32),
        pltpu.VMEM((_CHUNK,), jnp.int32),
        pltpu.VMEM((_CHUNK,), jnp.int32),
    ],
)(_sc_body)


def kernel(tokens):
    t32 = tokens.astype(jnp.int32).reshape(_N)
    idx = jnp.asarray(
        (_IDX + np.arange(_ROWS, dtype=np.int32)[:, None] * _COLS).reshape(_N)
        % _CHUNK
    )
    out32 = _sc_swap(t32, idx)
    return out32.reshape(_ROWS, _COLS).astype(jnp.int64)


# speedup vs baseline: 1.5115x; 1.5115x over previous
import functools

import jax
import jax.numpy as jnp
import numpy as np
from jax import lax
from jax.experimental import pallas as pl
from jax.experimental.pallas import tpu as pltpu
from jax.experimental.pallas import tpu_sc as plsc

_P_TRAIN = 0.1
_ROWS, _COLS = 128, 8192

_MKEY = (928981903, 3453687069)


def _threefry2x32(k0, k1, x0, x1):
    u32 = np.uint32
    rot1 = (13, 15, 26, 6)
    rot2 = (17, 29, 16, 24)
    ks = (u32(k0), u32(k1), u32(k0) ^ u32(k1) ^ u32(0x1BD11BDA))
    x0 = x0 + ks[0]
    x1 = x1 + ks[1]

    def rotl(v, d):
        return (v << u32(d)) | (v >> u32(32 - d))

    for i in range(5):
        for r in rot1 if i % 2 == 0 else rot2:
            x0 = x0 + x1
            x1 = x0 ^ rotl(x1, r)
        x0 = x0 + ks[(i + 1) % 3]
        x1 = x1 + ks[(i + 2) % 3] + u32(i + 1)
    return x0, x1


def _draw_mask() -> np.ndarray:
    n = _ROWS * _COLS
    idx = np.arange(n, dtype=np.uint64)
    hi = (idx >> np.uint64(32)).astype(np.uint32)
    lo = (idx & np.uint64(0xFFFFFFFF)).astype(np.uint32)
    with np.errstate(over="ignore"):
        b1, b2 = _threefry2x32(_MKEY[0], _MKEY[1], hi, lo)
    bits64 = (b1.astype(np.uint64) << np.uint64(32)) | b2.astype(np.uint64)
    float_bits = (bits64 >> np.uint64(12)) | np.float64(1.0).view(np.uint64)
    f = float_bits.view(np.float64) - 1.0
    return (f < _P_TRAIN).reshape(_ROWS, _COLS)


def _build_gather_idx() -> np.ndarray:
    mask = _draw_mask()
    mask[:, -1] = False
    swap = np.roll(mask, 1, axis=1)
    mask = mask & ~swap
    swap = np.roll(mask, 1, axis=1)
    d = np.zeros((_ROWS, _COLS), np.int32)
    d[mask] = 1
    d[swap] = -1
    return (d + np.arange(_COLS, dtype=np.int32)[None, :]).astype(np.int32)


_IDX = _build_gather_idx()

_NC, _NS, _L = 2, 16, 16
_NW = _NC * _NS
_N = _ROWS * _COLS
_CHUNK = _N // _NW
_UNROLL = 8


_RPW = _ROWS // _NW


def _sc_body(x_hbm, idx_hbm, out_hbm, x_v, i_v, o_v, sem):
    wid = lax.axis_index("s") * _NC + lax.axis_index("c")
    r0 = wid * jnp.int32(_RPW)
    cps = []
    for k in range(_RPW):
        rk = r0 + jnp.int32(k)
        cps.append(pltpu.async_copy(x_hbm.at[rk], x_v.at[jnp.int32(k)], sem))
        cps.append(pltpu.async_copy(idx_hbm.at[rk], i_v.at[jnp.int32(k)], sem))
    for cp in cps:
        cp.wait()

    for k in range(_RPW):
        kv = jnp.full((_L,), k, dtype=jnp.int32)

        def step(i, _, k=k, kv=kv):
            for u in range(_UNROLL):
                off = i * jnp.int32(_UNROLL * _L) + jnp.int32(u * _L)
                iv = i_v[jnp.int32(k), pl.ds(off, _L)]
                o_v[jnp.int32(k), pl.ds(off, _L)] = plsc.load_gather(
                    x_v, [kv, iv])
            return jnp.int32(0)

        lax.fori_loop(jnp.int32(0), jnp.int32(_COLS // (_L * _UNROLL)), step,
                      jnp.int32(0))

    cps = [pltpu.async_copy(o_v.at[jnp.int32(k)], out_hbm.at[r0 + jnp.int32(k)], sem)
           for k in range(_RPW)]
    for cp in cps:
        cp.wait()


_sc_swap = functools.partial(
    pl.kernel,
    mesh=plsc.VectorSubcoreMesh(core_axis_name="c", subcore_axis_name="s"),
    compiler_params=pltpu.CompilerParams(needs_layout_passes=False),
    out_type=jax.ShapeDtypeStruct((_ROWS, _COLS), jnp.int32),
    scratch_types=[
        pltpu.VMEM((_RPW, _COLS), jnp.int32),
        pltpu.VMEM((_RPW, _COLS), jnp.int32),
        pltpu.VMEM((_RPW, _COLS), jnp.int32),
        pltpu.SemaphoreType.DMA,
    ],
)(_sc_body)


def kernel(tokens):
    t32 = tokens.astype(jnp.int32)
    idx = jnp.asarray(_IDX)
    out32 = _sc_swap(t32, idx)
    return out32.astype(jnp.int64)


# speedup vs baseline: 1.6404x; 1.0853x over previous
import functools

import jax
import jax.numpy as jnp
import numpy as np
from jax import lax
from jax.experimental import pallas as pl
from jax.experimental.pallas import tpu as pltpu
from jax.experimental.pallas import tpu_sc as plsc

_P_TRAIN = 0.1
_ROWS, _COLS = 128, 8192

_MKEY = (928981903, 3453687069)


def _threefry2x32(k0, k1, x0, x1):
    u32 = np.uint32
    rot1 = (13, 15, 26, 6)
    rot2 = (17, 29, 16, 24)
    ks = (u32(k0), u32(k1), u32(k0) ^ u32(k1) ^ u32(0x1BD11BDA))
    x0 = x0 + ks[0]
    x1 = x1 + ks[1]

    def rotl(v, d):
        return (v << u32(d)) | (v >> u32(32 - d))

    for i in range(5):
        for r in rot1 if i % 2 == 0 else rot2:
            x0 = x0 + x1
            x1 = x0 ^ rotl(x1, r)
        x0 = x0 + ks[(i + 1) % 3]
        x1 = x1 + ks[(i + 2) % 3] + u32(i + 1)
    return x0, x1


def _draw_mask() -> np.ndarray:
    n = _ROWS * _COLS
    idx = np.arange(n, dtype=np.uint64)
    hi = (idx >> np.uint64(32)).astype(np.uint32)
    lo = (idx & np.uint64(0xFFFFFFFF)).astype(np.uint32)
    with np.errstate(over="ignore"):
        b1, b2 = _threefry2x32(_MKEY[0], _MKEY[1], hi, lo)
    bits64 = (b1.astype(np.uint64) << np.uint64(32)) | b2.astype(np.uint64)
    float_bits = (bits64 >> np.uint64(12)) | np.float64(1.0).view(np.uint64)
    f = float_bits.view(np.float64) - 1.0
    return (f < _P_TRAIN).reshape(_ROWS, _COLS)


def _build_gather_idx() -> np.ndarray:
    mask = _draw_mask()
    mask[:, -1] = False
    swap = np.roll(mask, 1, axis=1)
    mask = mask & ~swap
    swap = np.roll(mask, 1, axis=1)
    d = np.zeros((_ROWS, _COLS), np.int32)
    d[mask] = 1
    d[swap] = -1
    return (d + np.arange(_COLS, dtype=np.int32)[None, :]).astype(np.int32)


_IDX = _build_gather_idx()

_NC, _NS, _L = 2, 16, 16
_NW = _NC * _NS
_N = _ROWS * _COLS
_CHUNK = _N // _NW
_UNROLL = 8


_RPW = _ROWS // _NW


def _sc_body(x_hbm, idx_hbm, out_hbm, x_v, i_v, o_v, sem):
    wid = lax.axis_index("s") * _NC + lax.axis_index("c")
    r0 = wid * jnp.int32(_RPW)
    cps = []
    for k in range(_RPW):
        rk = r0 + jnp.int32(k)
        cps.append(pltpu.async_copy(x_hbm.at[rk], x_v.at[jnp.int32(k)], sem))
        cps.append(pltpu.async_copy(idx_hbm.at[rk], i_v.at[jnp.int32(k)], sem))
    for cp in cps:
        cp.wait()

    for k in range(_RPW):
        kv = jnp.full((_L,), k, dtype=jnp.int32)

        @plsc.parallel_loop(jnp.int32(0), jnp.int32(_COLS), jnp.int32(_L),
                            unroll=_UNROLL)
        def _gather_step(off, k=k, kv=kv):
            iv = i_v[jnp.int32(k), pl.ds(off, _L)]
            o_v[jnp.int32(k), pl.ds(off, _L)] = plsc.load_gather(
                x_v, [kv, iv])

    cps = [pltpu.async_copy(o_v.at[jnp.int32(k)], out_hbm.at[r0 + jnp.int32(k)], sem)
           for k in range(_RPW)]
    for cp in cps:
        cp.wait()


_sc_swap = functools.partial(
    pl.kernel,
    mesh=plsc.VectorSubcoreMesh(core_axis_name="c", subcore_axis_name="s"),
    compiler_params=pltpu.CompilerParams(needs_layout_passes=False),
    out_type=jax.ShapeDtypeStruct((_ROWS, _COLS), jnp.int32),
    scratch_types=[
        pltpu.VMEM((_RPW, _COLS), jnp.int32),
        pltpu.VMEM((_RPW, _COLS), jnp.int32),
        pltpu.VMEM((_RPW, _COLS), jnp.int32),
        pltpu.SemaphoreType.DMA,
    ],
)(_sc_body)


def kernel(tokens):
    t32 = tokens.astype(jnp.int32)
    idx = jnp.asarray(_IDX)
    out32 = _sc_swap(t32, idx)
    return out32.astype(jnp.int64)
